# baseline (speedup 1.0000x reference)
import functools

import jax
import jax.numpy as jnp
from jax import lax
from jax.experimental import pallas as pl
from jax.experimental.pallas import tpu as pltpu
from jax.experimental.pallas import tpu_sc as plsc

_NC = 2
_NS = 16
_NW = _NC * _NS
_LANES = 16

_CHUNK = 128
_NBUF = 8


@functools.lru_cache(maxsize=None)
def _make_gather(B, F, V, D):
  n = B * F
  rows_w = n // _NW
  nch = rows_w // _CHUNK
  ngrp = nch // _NBUF
  assert rows_w % (_CHUNK * _NBUF) == 0 and _CHUNK % _LANES == 0

  mesh = plsc.VectorSubcoreMesh(
      core_axis_name="c", subcore_axis_name="s",
      num_cores=_NC, num_subcores=_NS)

  @functools.partial(
      pl.kernel,
      out_type=jax.ShapeDtypeStruct((n, D), jnp.float32),
      mesh=mesh,
      scratch_types=[
          pltpu.VMEM((nch, _CHUNK), jnp.int32),
          *[pltpu.VMEM((_CHUNK, D), jnp.float32) for _ in range(_NBUF)],
          *[pltpu.SemaphoreType.DMA for _ in range(_NBUF)],
      ],
      compiler_params=pltpu.CompilerParams(use_tc_tiling_on_sc=False),
  )
  def gather_kernel(xin_hbm, table_hbm, out_hbm, idx_v, *bufs_sems):
    bufs = bufs_sems[:_NBUF]
    sems = bufs_sems[_NBUF:]
    wid = lax.axis_index("s") * _NC + lax.axis_index("c")
    base = wid * rows_w

    pltpu.sync_copy(xin_hbm.at[wid], idx_v)

    vecs_per_chunk = _CHUNK // _LANES

    @pl.loop(0, nch)
    def _fix(c):
      for j in range(vecs_per_chunk):
        pos = base + c * _CHUNK + j * _LANES + lax.iota(jnp.int32, _LANES)
        f = lax.rem(pos, F)
        sl = pl.ds(j * _LANES, _LANES)
        idx_v[c, sl] = idx_v[c, sl] + f * V

    def start(c, slot):
      pltpu.async_copy(table_hbm.at[idx_v.at[c]], bufs[slot], sems[slot])

    def drain(c, slot):
      pltpu.make_async_copy(
          table_hbm.at[idx_v.at[c]], bufs[slot], sems[slot]).wait()
      pltpu.sync_copy(bufs[slot], out_hbm.at[pl.ds(base + c * _CHUNK, _CHUNK)])

    for b in range(_NBUF):
      start(b, b)

    @pl.loop(0, ngrp - 1)
    def _grp(g):
      c0 = g * _NBUF
      for b in range(_NBUF):
        drain(c0 + b, b)
        start(c0 + _NBUF + b, b)

    c0 = (ngrp - 1) * _NBUF
    for b in range(_NBUF):
      drain(c0 + b, b)

  return gather_kernel


def _stats_body(x_ref, o_ref):
  @pl.when(pl.program_id(0) == 0)
  def _():
    o_ref[...] = jnp.zeros_like(o_ref)

  xb = x_ref[...]
  o_ref[0:1, :] += jnp.sum(xb, axis=0, keepdims=True)
  o_ref[1:2, :] += jnp.sum(xb * xb, axis=0, keepdims=True)


def _head_body(nb_inv, x_ref, st_ref, g_ref, be_ref, w_ref, b_ref, o_ref):
  mean = st_ref[0:1, :] * nb_inv
  var = st_ref[1:2, :] * nb_inv - mean * mean
  scale = g_ref[...] * lax.rsqrt(var + 1e-5)
  shift = be_ref[...] - mean * scale
  xn = x_ref[...] * scale + shift
  y = lax.dot_general(xn, w_ref[...], (((1,), (1,)), ((), ())),
                      preferred_element_type=jnp.float32)
  o_ref[...] = jnp.maximum(y + b_ref[...], 0.0)


@functools.lru_cache(maxsize=None)
def _make_head(B, K, OUT, blk):
  nb = B // blk
  stats = pl.pallas_call(
      _stats_body,
      grid=(nb,),
      in_specs=[pl.BlockSpec((blk, K), lambda i: (i, 0))],
      out_specs=pl.BlockSpec((2, K), lambda i: (0, 0)),
      out_shape=jax.ShapeDtypeStruct((2, K), jnp.float32),
  )
  head = pl.pallas_call(
      functools.partial(_head_body, 1.0 / B),
      grid=(nb,),
      in_specs=[
          pl.BlockSpec((blk, K), lambda i: (i, 0)),
          pl.BlockSpec((2, K), lambda i: (0, 0)),
          pl.BlockSpec((1, K), lambda i: (0, 0)),
          pl.BlockSpec((1, K), lambda i: (0, 0)),
          pl.BlockSpec((OUT, K), lambda i: (0, 0)),
          pl.BlockSpec((1, OUT), lambda i: (0, 0)),
      ],
      out_specs=pl.BlockSpec((blk, OUT), lambda i: (i, 0)),
      out_shape=jax.ShapeDtypeStruct((B, OUT), jnp.float32),
  )
  return stats, head


def kernel(x_in, emb_tables, bn_gamma, bn_beta, W, b):
  B, F = x_in.shape
  _, V, D = emb_tables.shape
  OUT = W.shape[0]
  K = F * D

  n = B * F
  rows_w = n // _NW
  xin3 = x_in.reshape(_NW, rows_w // _CHUNK, _CHUNK)
  table_flat = emb_tables.reshape(F * V, D)

  gathered = _make_gather(B, F, V, D)(xin3, table_flat)
  return gathered
  x2d = gathered.reshape(B, K)

  stats_call, head_call = _make_head(B, K, OUT, 2048)
  st = stats_call(x2d)
  return head_call(x2d, st, bn_gamma.reshape(1, K), bn_beta.reshape(1, K),
                   W, b.reshape(1, OUT))


# speedup vs baseline: 1.1446x; 1.1446x over previous
import functools

import jax
import jax.numpy as jnp
from jax import lax
from jax.experimental import pallas as pl
from jax.experimental.pallas import tpu as pltpu
from jax.experimental.pallas import tpu_sc as plsc

_NC = 2
_NS = 16
_NW = _NC * _NS
_LANES = 16

_CHUNK = 128
_NBUF = 8


@functools.lru_cache(maxsize=None)
def _make_gather(B, F, V, D):
  n = B * F
  rows_w = n // _NW
  nch = rows_w // _CHUNK
  ngrp = nch // _NBUF
  nj = -(-F * D // 128)
  fields_per_plane = 128 // D
  assert rows_w % (_CHUNK * _NBUF) == 0 and B % _CHUNK == 0

  mesh = plsc.VectorSubcoreMesh(
      core_axis_name="c", subcore_axis_name="s",
      num_cores=_NC, num_subcores=_NS)

  @functools.partial(
      pl.kernel,
      out_type=jax.ShapeDtypeStruct((nj, B, 128), jnp.float32),
      mesh=mesh,
      scratch_types=[
          pltpu.VMEM((nch, _CHUNK), jnp.int32),
          *[pltpu.VMEM((_CHUNK, D), jnp.float32) for _ in range(_NBUF)],
          *[pltpu.SemaphoreType.DMA for _ in range(_NBUF)],
      ],
      compiler_params=pltpu.CompilerParams(use_tc_tiling_on_sc=False),
  )
  def gather_kernel(xin_hbm, table_hbm, out_hbm, idx_v, *bufs_sems):
    bufs = bufs_sems[:_NBUF]
    sems = bufs_sems[_NBUF:]
    wid = lax.axis_index("s") * _NC + lax.axis_index("c")
    base = wid * rows_w

    pltpu.sync_copy(xin_hbm.at[pl.ds(wid * nch, nch)], idx_v)

    @pl.loop(0, nch)
    def _fix(c):
      f = (base + c * _CHUNK) // B
      off = f * V
      for jj in range(_CHUNK // _LANES):
        sl = pl.ds(jj * _LANES, _LANES)
        idx_v[c, sl] = idx_v[c, sl] + off

    def start(c, slot):
      pltpu.async_copy(table_hbm.at[idx_v.at[c]], bufs[slot], sems[slot])

    def drain(c, slot):
      pltpu.make_async_copy(
          table_hbm.at[idx_v.at[c]], bufs[slot], sems[slot]).wait()
      g0 = base + c * _CHUNK
      f = g0 // B
      b0 = g0 - f * B
      j = f // fields_per_plane
      c0 = (f % fields_per_plane) * D
      pltpu.sync_copy(bufs[slot],
                      out_hbm.at[j, pl.ds(b0, _CHUNK), pl.ds(c0, D)])

    for b in range(_NBUF):
      start(b, b)

    @pl.loop(0, ngrp - 1)
    def _grp(g):
      c0 = g * _NBUF
      for b in range(_NBUF):
        drain(c0 + b, b)
        start(c0 + _NBUF + b, b)

    c0 = (ngrp - 1) * _NBUF
    for b in range(_NBUF):
      drain(c0 + b, b)

  return gather_kernel


def _masked(xb, nvalid):
  if nvalid == 128:
    return xb
  lane = lax.broadcasted_iota(jnp.int32, xb.shape, xb.ndim - 1)
  return jnp.where(lane < nvalid, xb, 0.0)


def _stats_body(nvalid_last, x_ref, o_ref):
  @pl.when(pl.program_id(0) == 0)
  def _():
    o_ref[...] = jnp.zeros_like(o_ref)

  nj = x_ref.shape[0]
  for j in range(nj):
    xb = x_ref[j]
    if j == nj - 1:
      xb = _masked(xb, nvalid_last)
    o_ref[j:j + 1, :] += jnp.sum(xb, axis=0, keepdims=True)
    o_ref[nj + j:nj + j + 1, :] += jnp.sum(xb * xb, axis=0, keepdims=True)


def _head_body(nb_inv, nvalid_last, x_ref, st_ref, g_ref, be_ref, w_ref,
               b_ref, o_ref):
  nj = x_ref.shape[0]
  acc = None
  for j in range(nj):
    mean = st_ref[j:j + 1, :] * nb_inv
    var = st_ref[nj + j:nj + j + 1, :] * nb_inv - mean * mean
    scale = g_ref[j:j + 1, :] * lax.rsqrt(var + 1e-5)
    shift = be_ref[j:j + 1, :] - mean * scale
    xb = x_ref[j]
    if j == nj - 1:
      xb = _masked(xb, nvalid_last)
      shift = _masked(shift, nvalid_last)
    xn = xb * scale + shift
    y = lax.dot_general(xn, w_ref[j], (((1,), (0,)), ((), ())),
                        preferred_element_type=jnp.float32)
    acc = y if acc is None else acc + y
  o_ref[...] = jnp.maximum(acc + b_ref[...], 0.0)


@functools.lru_cache(maxsize=None)
def _make_head(B, K, OUT, nj, blk):
  nb = B // blk
  nvalid_last = K - 128 * (nj - 1)
  stats = pl.pallas_call(
      functools.partial(_stats_body, nvalid_last),
      grid=(nb,),
      in_specs=[pl.BlockSpec((nj, blk, 128), lambda i: (0, i, 0))],
      out_specs=pl.BlockSpec((2 * nj, 128), lambda i: (0, 0)),
      out_shape=jax.ShapeDtypeStruct((2 * nj, 128), jnp.float32),
  )
  head = pl.pallas_call(
      functools.partial(_head_body, 1.0 / B, nvalid_last),
      grid=(nb,),
      in_specs=[
          pl.BlockSpec((nj, blk, 128), lambda i: (0, i, 0)),
          pl.BlockSpec((2 * nj, 128), lambda i: (0, 0)),
          pl.BlockSpec((nj, 128), lambda i: (0, 0)),
          pl.BlockSpec((nj, 128), lambda i: (0, 0)),
          pl.BlockSpec((nj, 128, OUT), lambda i: (0, 0, 0)),
          pl.BlockSpec((1, OUT), lambda i: (0, 0)),
      ],
      out_specs=pl.BlockSpec((blk, OUT), lambda i: (i, 0)),
      out_shape=jax.ShapeDtypeStruct((B, OUT), jnp.float32),
  )
  return stats, head


def kernel(x_in, emb_tables, bn_gamma, bn_beta, W, b):
  B, F = x_in.shape
  _, V, D = emb_tables.shape
  OUT = W.shape[0]
  K = F * D
  nj = -(-K // 128)
  kpad = nj * 128

  xin2 = x_in.T.reshape(B * F // 128, 128)
  table_flat = emb_tables.reshape(F * V, D)

  x4 = _make_gather(B, F, V, D)(xin2, table_flat)

  w4 = jnp.transpose(
      jnp.pad(W, ((0, 0), (0, kpad - K))).reshape(OUT, nj, 128), (1, 2, 0))
  g4 = jnp.pad(bn_gamma, (0, kpad - K)).reshape(nj, 128)
  be4 = jnp.pad(bn_beta, (0, kpad - K)).reshape(nj, 128)

  stats_call, head_call = _make_head(B, K, OUT, nj, 2048)
  st = stats_call(x4)
  return head_call(x4, st, g4, be4, w4, b.reshape(1, OUT))


# speedup vs baseline: 1.1446x; 1.0000x over previous
import functools

import jax
import jax.numpy as jnp
from jax import lax
from jax.experimental import pallas as pl
from jax.experimental.pallas import tpu as pltpu
from jax.experimental.pallas import tpu_sc as plsc

_NC = 2
_NS = 16
_NW = _NC * _NS
_LANES = 16

_CHUNK = 128
_NBUF = 8


def _mesh():
  return plsc.VectorSubcoreMesh(
      core_axis_name="c", subcore_axis_name="s",
      num_cores=_NC, num_subcores=_NS)


@functools.lru_cache(maxsize=None)
def _make_index(B, F, V):
  assert F <= _NW

  @functools.partial(
      pl.kernel,
      out_type=jax.ShapeDtypeStruct((F * B,), jnp.int32),
      mesh=_mesh(),
      scratch_types=[pltpu.VMEM((B,), jnp.int32)],
  )
  def index_kernel(xt_hbm, out_hbm, ids_v):
    wid = lax.axis_index("s") * _NC + lax.axis_index("c")

    @pl.when(wid < F)
    def _():
      pltpu.sync_copy(xt_hbm.at[wid], ids_v)
      off = wid * V

      @pl.loop(0, B // _LANES)
      def _fix(k):
        sl = pl.ds(k * _LANES, _LANES)
        ids_v[sl] = ids_v[sl] + off

      pltpu.sync_copy(ids_v, out_hbm.at[pl.ds(wid * B, B)])

  return index_kernel


@functools.lru_cache(maxsize=None)
def _make_gather(B, F, V, D):
  nch = B // _CHUNK
  ngrp = nch // _NBUF
  nj = -(-F * D // 128)
  fields_per_plane = 128 // D
  assert B % (_CHUNK * _NBUF) == 0 and F <= _NW

  @functools.partial(
      pl.kernel,
      out_type=jax.ShapeDtypeStruct((nj, B, 128), jnp.float32),
      mesh=_mesh(),
      scratch_types=[
          pltpu.VMEM((nch, _CHUNK), jnp.int32),
          *[pltpu.VMEM((_CHUNK, D), jnp.float32) for _ in range(_NBUF)],
          *[pltpu.SemaphoreType.DMA for _ in range(_NBUF)],
      ],
      compiler_params=pltpu.CompilerParams(use_tc_tiling_on_sc=False),
  )
  def gather_kernel(idx_hbm, table_hbm, out_hbm, idx_v, *bufs_sems):
    bufs = bufs_sems[:_NBUF]
    sems = bufs_sems[_NBUF:]
    wid = lax.axis_index("s") * _NC + lax.axis_index("c")

    @pl.when(wid < F)
    def _():
      j = wid // fields_per_plane
      c0 = (wid % fields_per_plane) * D
      pltpu.sync_copy(idx_hbm.at[pl.ds(wid * nch, nch)], idx_v)

      def start(c, slot):
        pltpu.async_copy(table_hbm.at[idx_v.at[c]], bufs[slot], sems[slot])

      def drain(c, slot):
        pltpu.make_async_copy(
            table_hbm.at[idx_v.at[c]], bufs[slot], sems[slot]).wait()
        pltpu.sync_copy(bufs[slot],
                        out_hbm.at[j, pl.ds(c * _CHUNK, _CHUNK),
                                   pl.ds(c0, D)])

      for b in range(_NBUF):
        start(b, b)

      @pl.loop(0, ngrp - 1)
      def _grp(g):
        cc = g * _NBUF
        for b in range(_NBUF):
          drain(cc + b, b)
          start(cc + _NBUF + b, b)

      cc = (ngrp - 1) * _NBUF
      for b in range(_NBUF):
        drain(cc + b, b)

  return gather_kernel


def _masked(xb, nvalid):
  if nvalid == 128:
    return xb
  lane = lax.broadcasted_iota(jnp.int32, xb.shape, xb.ndim - 1)
  return jnp.where(lane < nvalid, xb, 0.0)


def _stats_body(nvalid_last, x_ref, o_ref):
  @pl.when(pl.program_id(0) == 0)
  def _():
    o_ref[...] = jnp.zeros_like(o_ref)

  nj = x_ref.shape[0]
  for j in range(nj):
    xb = x_ref[j]
    if j == nj - 1:
      xb = _masked(xb, nvalid_last)
    o_ref[j:j + 1, :] += jnp.sum(xb, axis=0, keepdims=True)
    o_ref[nj + j:nj + j + 1, :] += jnp.sum(xb * xb, axis=0, keepdims=True)


def _head_body(nb_inv, nvalid_last, x_ref, st_ref, g_ref, be_ref, w_ref,
               b_ref, o_ref):
  nj = x_ref.shape[0]
  acc = None
  for j in range(nj):
    mean = st_ref[j:j + 1, :] * nb_inv
    var = st_ref[nj + j:nj + j + 1, :] * nb_inv - mean * mean
    scale = g_ref[j:j + 1, :] * lax.rsqrt(var + 1e-5)
    shift = be_ref[j:j + 1, :] - mean * scale
    xb = x_ref[j]
    if j == nj - 1:
      xb = _masked(xb, nvalid_last)
      shift = _masked(shift, nvalid_last)
    xn = xb * scale + shift
    y = lax.dot_general(xn, w_ref[j], (((1,), (1,)), ((), ())),
                        preferred_element_type=jnp.float32)
    acc = y if acc is None else acc + y
  o_ref[...] = jnp.maximum(acc + b_ref[...], 0.0)


@functools.lru_cache(maxsize=None)
def _make_head(B, K, OUT, nj, blk):
  nb = B // blk
  nvalid_last = K - 128 * (nj - 1)
  stats = pl.pallas_call(
      functools.partial(_stats_body, nvalid_last),
      grid=(nb,),
      in_specs=[pl.BlockSpec((nj, blk, 128), lambda i: (0, i, 0))],
      out_specs=pl.BlockSpec((2 * nj, 128), lambda i: (0, 0)),
      out_shape=jax.ShapeDtypeStruct((2 * nj, 128), jnp.float32),
  )
  head = pl.pallas_call(
      functools.partial(_head_body, 1.0 / B, nvalid_last),
      grid=(nb,),
      in_specs=[
          pl.BlockSpec((nj, blk, 128), lambda i: (0, i, 0)),
          pl.BlockSpec((2 * nj, 128), lambda i: (0, 0)),
          pl.BlockSpec((nj, 128), lambda i: (0, 0)),
          pl.BlockSpec((nj, 128), lambda i: (0, 0)),
          pl.BlockSpec((nj, OUT, 128), lambda i: (0, 0, 0)),
          pl.BlockSpec((1, OUT), lambda i: (0, 0)),
      ],
      out_specs=pl.BlockSpec((blk, OUT), lambda i: (i, 0)),
      out_shape=jax.ShapeDtypeStruct((B, OUT), jnp.float32),
  )
  return stats, head


def kernel(x_in, emb_tables, bn_gamma, bn_beta, W, b):
  B, F = x_in.shape
  _, V, D = emb_tables.shape
  OUT = W.shape[0]
  K = F * D
  nj = -(-K // 128)
  kpad = nj * 128

  xt = x_in.T
  table_flat = emb_tables.reshape(F * V, D)

  ids = _make_index(B, F, V)(xt)
  idx2 = ids.reshape(F * B // _CHUNK, _CHUNK)
  x4 = _make_gather(B, F, V, D)(idx2, table_flat)

  w4 = jnp.transpose(
      jnp.pad(W, ((0, 0), (0, kpad - K))).reshape(OUT, nj, 128), (1, 0, 2))
  g4 = jnp.pad(bn_gamma, (0, kpad - K)).reshape(nj, 128)
  be4 = jnp.pad(bn_beta, (0, kpad - K)).reshape(nj, 128)

  stats_call, head_call = _make_head(B, K, OUT, nj, 2048)
  st = stats_call(x4)
  return head_call(x4, st, g4, be4, w4, b.reshape(1, OUT))


# speedup vs baseline: 1.1447x; 1.0000x over previous
import functools

import jax
import jax.numpy as jnp
from jax import lax
from jax.experimental import pallas as pl
from jax.experimental.pallas import tpu as pltpu
from jax.experimental.pallas import tpu_sc as plsc

_NC = 2
_NS = 16
_NW = _NC * _NS
_LANES = 16

_CHUNK = 128
_NBUF = 8


def _mesh():
  return plsc.VectorSubcoreMesh(
      core_axis_name="c", subcore_axis_name="s",
      num_cores=_NC, num_subcores=_NS)


@functools.lru_cache(maxsize=None)
def _make_index(B, F, V):
  assert F <= _NW

  @functools.partial(
      pl.kernel,
      out_type=jax.ShapeDtypeStruct((F * B,), jnp.int32),
      mesh=_mesh(),
      scratch_types=[pltpu.VMEM((B,), jnp.int32)],
  )
  def index_kernel(xt_hbm, out_hbm, ids_v):
    wid = lax.axis_index("s") * _NC + lax.axis_index("c")

    @pl.when(wid < F)
    def _():
      pltpu.sync_copy(xt_hbm.at[wid], ids_v)
      pltpu.sync_copy(ids_v, out_hbm.at[pl.ds(wid * B, B)])

  return index_kernel


@functools.lru_cache(maxsize=None)
def _make_gather(B, F, V, D):
  nch = B // _CHUNK
  ngrp = nch // _NBUF
  nj = -(-F * D // 128)
  fields_per_plane = 128 // D
  assert B % (_CHUNK * _NBUF) == 0 and F <= _NW

  @functools.partial(
      pl.kernel,
      out_type=jax.ShapeDtypeStruct((nj, B, 128), jnp.float32),
      mesh=_mesh(),
      scratch_types=[
          pltpu.VMEM((nch, _CHUNK), jnp.int32),
          *[pltpu.VMEM((_CHUNK, D), jnp.float32) for _ in range(_NBUF)],
          *[pltpu.SemaphoreType.DMA for _ in range(_NBUF)],
      ],
      compiler_params=pltpu.CompilerParams(use_tc_tiling_on_sc=False),
  )
  def gather_kernel(idx_hbm, table_hbm, out_hbm, idx_v, *bufs_sems):
    bufs = bufs_sems[:_NBUF]
    sems = bufs_sems[_NBUF:]
    wid = lax.axis_index("s") * _NC + lax.axis_index("c")

    @pl.when(wid < F)
    def _():
      j = wid // fields_per_plane
      c0 = (wid % fields_per_plane) * D
      ftbl = table_hbm.at[wid]
      pltpu.sync_copy(idx_hbm.at[pl.ds(wid * nch, nch)], idx_v)

      def start(c, slot):
        pltpu.async_copy(ftbl.at[idx_v.at[c]], bufs[slot], sems[slot])

      def drain(c, slot):
        pltpu.make_async_copy(
            ftbl.at[idx_v.at[c]], bufs[slot], sems[slot]).wait()
        pltpu.sync_copy(bufs[slot],
                        out_hbm.at[j, pl.ds(c * _CHUNK, _CHUNK),
                                   pl.ds(c0, D)])

      for b in range(_NBUF):
        start(b, b)

      @pl.loop(0, ngrp - 1)
      def _grp(g):
        cc = g * _NBUF
        for b in range(_NBUF):
          drain(cc + b, b)
          start(cc + _NBUF + b, b)

      cc = (ngrp - 1) * _NBUF
      for b in range(_NBUF):
        drain(cc + b, b)

  return gather_kernel


def _masked(xb, nvalid):
  if nvalid == 128:
    return xb
  lane = lax.broadcasted_iota(jnp.int32, xb.shape, xb.ndim - 1)
  return jnp.where(lane < nvalid, xb, 0.0)


def _stats_body(nvalid_last, x_ref, o_ref):
  @pl.when(pl.program_id(0) == 0)
  def _():
    o_ref[...] = jnp.zeros_like(o_ref)

  nj = x_ref.shape[0]
  for j in range(nj):
    xb = x_ref[j]
    if j == nj - 1:
      xb = _masked(xb, nvalid_last)
    o_ref[j:j + 1, :] += jnp.sum(xb, axis=0, keepdims=True)
    o_ref[nj + j:nj + j + 1, :] += jnp.sum(xb * xb, axis=0, keepdims=True)


def _head_body(nb_inv, nvalid_last, x_ref, st_ref, g_ref, be_ref, w_ref,
               b_ref, o_ref):
  nj = x_ref.shape[0]
  acc = None
  for j in range(nj):
    mean = st_ref[j:j + 1, :] * nb_inv
    var = st_ref[nj + j:nj + j + 1, :] * nb_inv - mean * mean
    scale = g_ref[j:j + 1, :] * lax.rsqrt(var + 1e-5)
    shift = be_ref[j:j + 1, :] - mean * scale
    xb = x_ref[j]
    if j == nj - 1:
      xb = _masked(xb, nvalid_last)
      shift = _masked(shift, nvalid_last)
    xn = xb * scale + shift
    y = lax.dot_general(xn, w_ref[j], (((1,), (1,)), ((), ())),
                        preferred_element_type=jnp.float32)
    acc = y if acc is None else acc + y
  o_ref[...] = jnp.maximum(acc + b_ref[...], 0.0)


@functools.lru_cache(maxsize=None)
def _make_head(B, K, OUT, nj, blk):
  nb = B // blk
  nvalid_last = K - 128 * (nj - 1)
  stats = pl.pallas_call(
      functools.partial(_stats_body, nvalid_last),
      grid=(nb,),
      in_specs=[pl.BlockSpec((nj, blk, 128), lambda i: (0, i, 0))],
      out_specs=pl.BlockSpec((2 * nj, 128), lambda i: (0, 0)),
      out_shape=jax.ShapeDtypeStruct((2 * nj, 128), jnp.float32),
  )
  head = pl.pallas_call(
      functools.partial(_head_body, 1.0 / B, nvalid_last),
      grid=(nb,),
      in_specs=[
          pl.BlockSpec((nj, blk, 128), lambda i: (0, i, 0)),
          pl.BlockSpec((2 * nj, 128), lambda i: (0, 0)),
          pl.BlockSpec((nj, 128), lambda i: (0, 0)),
          pl.BlockSpec((nj, 128), lambda i: (0, 0)),
          pl.BlockSpec((nj, OUT, 128), lambda i: (0, 0, 0)),
          pl.BlockSpec((1, OUT), lambda i: (0, 0)),
      ],
      out_specs=pl.BlockSpec((blk, OUT), lambda i: (i, 0)),
      out_shape=jax.ShapeDtypeStruct((B, OUT), jnp.float32),
  )
  return stats, head


def kernel(x_in, emb_tables, bn_gamma, bn_beta, W, b):
  B, F = x_in.shape
  _, V, D = emb_tables.shape
  OUT = W.shape[0]
  K = F * D
  nj = -(-K // 128)
  kpad = nj * 128

  xt = x_in.T

  ids = _make_index(B, F, V)(xt)
  idx2 = ids.reshape(F * B // _CHUNK, _CHUNK)
  x4 = _make_gather(B, F, V, D)(idx2, emb_tables)

  w4 = jnp.transpose(
      jnp.pad(W, ((0, 0), (0, kpad - K))).reshape(OUT, nj, 128), (1, 0, 2))
  g4 = jnp.pad(bn_gamma, (0, kpad - K)).reshape(nj, 128)
  be4 = jnp.pad(bn_beta, (0, kpad - K)).reshape(nj, 128)

  stats_call, head_call = _make_head(B, K, OUT, nj, 2048)
  st = stats_call(x4)
  return head_call(x4, st, g4, be4, w4, b.reshape(1, OUT))


# speedup vs baseline: 4.6862x; 4.0940x over previous
import functools

import jax
import jax.numpy as jnp
from jax import lax
from jax.experimental import pallas as pl
from jax.experimental.pallas import tpu as pltpu
from jax.experimental.pallas import tpu_sc as plsc

_NC = 2
_NS = 16
_NW = _NC * _NS
_LANES = 16


def _mesh():
  return plsc.VectorSubcoreMesh(
      core_axis_name="c", subcore_axis_name="s",
      num_cores=_NC, num_subcores=_NS)


@functools.lru_cache(maxsize=None)
def _make_index(B, F):
  assert F <= _NW

  @functools.partial(
      pl.kernel,
      out_type=jax.ShapeDtypeStruct((F * B,), jnp.int32),
      mesh=_mesh(),
      scratch_types=[pltpu.VMEM((B,), jnp.int32)],
  )
  def index_kernel(xt_hbm, out_hbm, ids_v):
    wid = lax.axis_index("s") * _NC + lax.axis_index("c")

    @pl.when(wid < F)
    def _():
      pltpu.sync_copy(xt_hbm.at[wid], ids_v)
      pltpu.sync_copy(ids_v, out_hbm.at[pl.ds(wid * B, B)])

  return index_kernel


@functools.lru_cache(maxsize=None)
def _make_gather(B, F, V, D):
  npairs = F * D
  per_w = -(-npairs // _NW)
  brows = B // 128
  hrows = brows // 2
  assert B % 256 == 0

  @functools.partial(
      pl.kernel,
      out_type=jax.ShapeDtypeStruct((npairs, brows, 128), jnp.float32),
      mesh=_mesh(),
      scratch_types=[
          pltpu.VMEM((V,), jnp.float32),
          pltpu.VMEM((hrows, 128), jnp.int32),
          pltpu.VMEM((brows, 128), jnp.float32),
      ],
      compiler_params=pltpu.CompilerParams(
          use_tc_tiling_on_sc=True, needs_layout_passes=False),
  )
  def gather_kernel(idx_hbm, tbl_hbm, out_hbm, row_v, ids_v, out_v):
    wid = lax.axis_index("s") * _NC + lax.axis_index("c")

    @pl.loop(0, per_w)
    def _pair(i):
      p = wid * per_w + i

      @pl.when(p < npairs)
      def _():
        f = p // D
        d = p - f * D
        pltpu.sync_copy(tbl_hbm.at[f, d], row_v)
        for h in range(2):
          pltpu.sync_copy(
              idx_hbm.at[pl.ds(f * brows + h * hrows, hrows)], ids_v)

          @pl.loop(0, hrows * 8)
          def _g(k):
            rr = k // 8
            l = (k - rr * 8) * _LANES
            ids = ids_v[rr, pl.ds(l, _LANES)]
            vals = plsc.load_gather(row_v, [ids])
            out_v[h * hrows + rr, pl.ds(l, _LANES)] = vals

        pltpu.sync_copy(out_v, out_hbm.at[p])

  return gather_kernel


def _stats_body(x_ref, o_ref):
  @pl.when(pl.program_id(0) == 0)
  def _():
    o_ref[...] = jnp.zeros_like(o_ref)

  xb = x_ref[...]
  s = jnp.sum(jnp.sum(xb, axis=1), axis=1, keepdims=True)
  sq = jnp.sum(jnp.sum(xb * xb, axis=1), axis=1, keepdims=True)
  o_ref[:, 0:1] += s
  o_ref[:, 1:2] += sq


def _head_body(nb_inv, rb, x_ref, st_ref, g_ref, be_ref, w_ref, b_ref, o_ref):
  mean = st_ref[:, 0:1] * nb_inv
  var = st_ref[:, 1:2] * nb_inv - mean * mean
  scale = g_ref[...] * lax.rsqrt(var + 1e-5)
  shift = be_ref[...] - mean * scale
  for r in range(rb):
    xn = x_ref[:, r, :] * scale + shift
    y = lax.dot_general(xn, w_ref[...], (((0,), (1,)), ((), ())),
                        preferred_element_type=jnp.float32)
    o_ref[pl.ds(r * 128, 128), :] = jnp.maximum(y + b_ref[...], 0.0)


@functools.lru_cache(maxsize=None)
def _make_head(B, K, OUT, rb):
  brows = B // 128
  nb = brows // rb
  stats = pl.pallas_call(
      _stats_body,
      grid=(nb,),
      in_specs=[pl.BlockSpec((K, rb, 128), lambda i: (0, i, 0))],
      out_specs=pl.BlockSpec((K, 2), lambda i: (0, 0)),
      out_shape=jax.ShapeDtypeStruct((K, 2), jnp.float32),
  )
  head = pl.pallas_call(
      functools.partial(_head_body, 1.0 / B, rb),
      grid=(nb,),
      in_specs=[
          pl.BlockSpec((K, rb, 128), lambda i: (0, i, 0)),
          pl.BlockSpec((K, 2), lambda i: (0, 0)),
          pl.BlockSpec((K, 1), lambda i: (0, 0)),
          pl.BlockSpec((K, 1), lambda i: (0, 0)),
          pl.BlockSpec((OUT, K), lambda i: (0, 0)),
          pl.BlockSpec((1, OUT), lambda i: (0, 0)),
      ],
      out_specs=pl.BlockSpec((rb * 128, OUT), lambda i: (i, 0)),
      out_shape=jax.ShapeDtypeStruct((B, OUT), jnp.float32),
  )
  return stats, head


def kernel(x_in, emb_tables, bn_gamma, bn_beta, W, b):
  B, F = x_in.shape
  _, V, D = emb_tables.shape
  OUT = W.shape[0]
  K = F * D

  xt = x_in.T
  tblT = emb_tables.transpose(0, 2, 1)

  ids = _make_index(B, F)(xt)
  idx2 = ids.reshape(F * B // 128, 128)
  x3 = _make_gather(B, F, V, D)(idx2, tblT)

  stats_call, head_call = _make_head(B, K, OUT, 8)
  st = stats_call(x3)
  return head_call(x3, st, bn_gamma.reshape(K, 1), bn_beta.reshape(K, 1),
                   W, b.reshape(1, OUT))


# speedup vs baseline: 5.7931x; 1.2362x over previous
import functools

import jax
import jax.numpy as jnp
from jax import lax
from jax.experimental import pallas as pl
from jax.experimental.pallas import tpu as pltpu
from jax.experimental.pallas import tpu_sc as plsc

_NC = 2
_NS = 16
_NW = _NC * _NS
_LANES = 16

_RSTREAMS = 4
_QUARters = 4


@functools.lru_cache(maxsize=None)
def _make_gather(B, F, V, D):
  npairs = F * D
  per_w = npairs // _NW
  brows = B // 128
  qrows = brows // _QUARters
  qids = B // _QUARters
  rstep = (V // _RSTREAMS) // 128 * 128
  roffs = [r * rstep for r in range(_RSTREAMS)]
  rlens = [rstep] * (_RSTREAMS - 1) + [V - rstep * (_RSTREAMS - 1)]
  assert npairs % _NW == 0 and B % (128 * _QUARters) == 0

  mesh = plsc.VectorSubcoreMesh(
      core_axis_name="c", subcore_axis_name="s",
      num_cores=_NC, num_subcores=_NS)

  @functools.partial(
      pl.kernel,
      out_type=jax.ShapeDtypeStruct((npairs, brows, 128), jnp.float32),
      mesh=mesh,
      scratch_types=[
          pltpu.VMEM((V,), jnp.float32),
          pltpu.VMEM((qids,), jnp.int32),
          pltpu.VMEM((qids,), jnp.int32),
          pltpu.VMEM((brows, 128), jnp.float32),
          pltpu.SemaphoreType.DMA,
          pltpu.SemaphoreType.DMA,
          pltpu.SemaphoreType.DMA,
          pltpu.SemaphoreType.DMA,
      ],
      compiler_params=pltpu.CompilerParams(
          use_tc_tiling_on_sc=True, needs_layout_passes=False),
  )
  def gather_kernel(xt_hbm, tbl_hbm, out_hbm, row_v, ids0, ids1, out_v,
                    sem_row, sem_i0, sem_i1, sem_out):
    wid = lax.axis_index("s") * _NC + lax.axis_index("c")
    idbuf = (ids0, ids1)
    idsem = (sem_i0, sem_i1)

    @pl.loop(0, per_w)
    def _pair(i):
      p = wid * per_w + i
      f = p // D
      d = p - f * D

      trow = tbl_hbm.at[f, d]
      idrow = xt_hbm.at[f]
      pltpu.async_copy(trow, row_v, sem_row)
      for q in range(2):
        pltpu.async_copy(idrow.at[pl.ds(q * qids, qids)], idbuf[q],
                         idsem[q])

      @pl.when(i > 0)
      def _():
        pltpu.make_async_copy(out_v, out_hbm.at[p - 1], sem_out).wait()

      pltpu.make_async_copy(trow, row_v, sem_row).wait()

      for q in range(_QUARters):
        buf = idbuf[q % 2]
        pltpu.make_async_copy(idrow.at[pl.ds(q * qids, qids)], buf,
                              idsem[q % 2]).wait()

        @pl.loop(0, qrows)
        def _g(rr):
          for jj in range(8):
            sl = pl.ds((rr * 8 + jj) * _LANES, _LANES)
            vals = plsc.load_gather(row_v, [buf[sl]])
            out_v[q * qrows + rr, pl.ds(jj * _LANES, _LANES)] = vals

        if q + 2 < _QUARters:
          pltpu.async_copy(
              idrow.at[pl.ds((q + 2) * qids, qids)], buf, idsem[q % 2])

      pltpu.async_copy(out_v, out_hbm.at[p], sem_out)

    pltpu.make_async_copy(
        out_v, out_hbm.at[wid * per_w + per_w - 1], sem_out).wait()

  return gather_kernel


def _stats_body(x_ref, o_ref):
  @pl.when(pl.program_id(0) == 0)
  def _():
    o_ref[...] = jnp.zeros_like(o_ref)

  xb = x_ref[...]
  s = jnp.sum(jnp.sum(xb, axis=1), axis=1, keepdims=True)
  sq = jnp.sum(jnp.sum(xb * xb, axis=1), axis=1, keepdims=True)
  o_ref[:, 0:1] += s
  o_ref[:, 1:2] += sq


def _head_body(nb_inv, rb, x_ref, st_ref, g_ref, be_ref, w_ref, b_ref, o_ref):
  mean = st_ref[:, 0:1] * nb_inv
  var = st_ref[:, 1:2] * nb_inv - mean * mean
  scale = g_ref[...] * lax.rsqrt(var + 1e-5)
  shift = be_ref[...] - mean * scale
  for r in range(rb):
    xn = x_ref[:, r, :] * scale + shift
    y = lax.dot_general(xn, w_ref[...], (((0,), (1,)), ((), ())),
                        preferred_element_type=jnp.float32)
    o_ref[pl.ds(r * 128, 128), :] = jnp.maximum(y + b_ref[...], 0.0)


@functools.lru_cache(maxsize=None)
def _make_head(B, K, OUT, rb):
  brows = B // 128
  nb = brows // rb
  stats = pl.pallas_call(
      _stats_body,
      grid=(nb,),
      in_specs=[pl.BlockSpec((K, rb, 128), lambda i: (0, i, 0))],
      out_specs=pl.BlockSpec((K, 2), lambda i: (0, 0)),
      out_shape=jax.ShapeDtypeStruct((K, 2), jnp.float32),
  )
  head = pl.pallas_call(
      functools.partial(_head_body, 1.0 / B, rb),
      grid=(nb,),
      in_specs=[
          pl.BlockSpec((K, rb, 128), lambda i: (0, i, 0)),
          pl.BlockSpec((K, 2), lambda i: (0, 0)),
          pl.BlockSpec((K, 1), lambda i: (0, 0)),
          pl.BlockSpec((K, 1), lambda i: (0, 0)),
          pl.BlockSpec((OUT, K), lambda i: (0, 0)),
          pl.BlockSpec((1, OUT), lambda i: (0, 0)),
      ],
      out_specs=pl.BlockSpec((rb * 128, OUT), lambda i: (i, 0)),
      out_shape=jax.ShapeDtypeStruct((B, OUT), jnp.float32),
  )
  return stats, head


def kernel(x_in, emb_tables, bn_gamma, bn_beta, W, b):
  B, F = x_in.shape
  _, V, D = emb_tables.shape
  OUT = W.shape[0]
  K = F * D

  xt = x_in.T
  tblT = emb_tables.transpose(0, 2, 1)

  x3 = _make_gather(B, F, V, D)(xt, tblT)

  stats_call, head_call = _make_head(B, K, OUT, 8)
  st = stats_call(x3)
  return head_call(x3, st, bn_gamma.reshape(K, 1), bn_beta.reshape(K, 1),
                   W, b.reshape(1, OUT))
